# baseline (speedup 1.0000x reference)
import functools

import jax
import jax.numpy as jnp
from jax import lax
from jax.experimental import pallas as pl
from jax.experimental.pallas import tpu as pltpu
from jax.experimental.pallas import tpu_sc as plsc

NUM_HASHES = 50000
HIDDEN = 768
PRIME = 31
BATCH = 4
SEQ = 8192

NC = 2
NS = 16
NW = NC * NS
TOK = BATCH * SEQ
T_PER_W = TOK // NW
CHUNK = 16
N_CHUNKS = T_PER_W // CHUNK
N2 = N_CHUNKS // 2
W_PER_ROW = SEQ // T_PER_W
VPR = HIDDEN // 16


def _hash_body(bytes_hbm, t3, t4, t5, out_hbm, vb,
               i3a, i4a, i5a, ga0a, ga1a, b4a, b5a,
               i3b, i4b, i5b, ga0b, ga1b, b4b, b5b,
               sga, soa, sgb, sob):
    wid = lax.axis_index("s") * NC + lax.axis_index("c")
    p0 = (wid % W_PER_ROW) * T_PER_W
    f0 = wid * T_PER_W

    zero16 = jnp.zeros((16,), jnp.int32)
    vb[pl.ds(0, 16)] = zero16

    @pl.when(f0 == 0)
    def _():
        pltpu.sync_copy(bytes_hbm.at[pl.ds(0, T_PER_W)],
                        vb.at[pl.ds(8, T_PER_W)])

    @pl.when(f0 != 0)
    def _():
        pltpu.sync_copy(bytes_hbm.at[pl.ds(f0 - 8, T_PER_W + 8)], vb)

    lanes = lax.iota(jnp.int32, 16)

    def hash_chunk(c, i3, i4, i5):
        for g in range(CHUNK // 16):
            o = c * CHUNK + g * 16
            b0 = vb[pl.ds(8 + o, 16)]
            b1 = vb[pl.ds(7 + o, 16)]
            b2 = vb[pl.ds(6 + o, 16)]
            b3 = vb[pl.ds(5 + o, 16)]
            b4 = vb[pl.ds(4 + o, 16)]

            t_in_row = lanes + (p0 + o)

            h = b2 * PRIME + b1
            h3 = (h * PRIME + b0) % NUM_HASHES
            h3 = jnp.where(t_in_row >= 2, h3, 0)

            h = (b3 * PRIME + b2)
            h = (h * PRIME + b1) % NUM_HASHES
            h4 = (h * PRIME + b0) % NUM_HASHES
            h4 = jnp.where(t_in_row >= 3, h4, 0)

            h = (b4 * PRIME + b3)
            h = (h * PRIME + b2) % NUM_HASHES
            h = (h * PRIME + b1) % NUM_HASHES
            h5 = (h * PRIME + b0) % NUM_HASHES
            h5 = jnp.where(t_in_row >= 4, h5, 0)

            i3[pl.ds(g * 16, 16)] = h3
            i4[pl.ds(g * 16, 16)] = h4
            i5[pl.ds(g * 16, 16)] = h5

    def fire3(i3, i4, i5, acc, b4, b5, sg):
        pltpu.async_copy(t3.at[i3], acc, sg)
        pltpu.async_copy(t4.at[i4], b4, sg)
        pltpu.async_copy(t5.at[i5], b5, sg)

    def wait3(i3, acc, b4, b5, sg):
        pltpu.make_async_copy(t3.at[i3], acc, sg).wait()
        pltpu.make_async_copy(t3.at[i3], b4, sg).wait()
        pltpu.make_async_copy(t3.at[i3], b5, sg).wait()

    def add2(acc, b4, b5):
        def add_row(r, carry):
            for u in range(VPR):
                plsc.addupdate(acc.at[r, pl.ds(u * 16, 16)],
                               b4[r, pl.ds(u * 16, 16)])
                plsc.addupdate(acc.at[r, pl.ds(u * 16, 16)],
                               b5[r, pl.ds(u * 16, 16)])
            return carry
        lax.fori_loop(0, CHUNK, add_row, 0)

    def scatter(acc, so, c):
        pltpu.async_copy(acc, out_hbm.at[pl.ds(f0 + c * CHUNK, CHUNK)], so)

    def waitsc(acc, so):
        pltpu.make_async_copy(acc, out_hbm.at[pl.ds(0, CHUNK)], so).wait()

    def phase(c, i3, i4, i5, gacc, ganext, b4, b5, sg, so,
              do_waitsc, do_fire):
        wait3(i3, gacc, b4, b5, sg)
        add2(gacc, b4, b5)
        scatter(gacc, so, c)

        @pl.when(do_waitsc)
        def _():
            waitsc(gacc, so)

        @pl.when(do_fire)
        def _():
            hash_chunk(c + 2, i3, i4, i5)
            fire3(i3, i4, i5, ganext, b4, b5, sg)

    hash_chunk(0, i3a, i4a, i5a)
    fire3(i3a, i4a, i5a, ga0a, b4a, b5a, sga)
    hash_chunk(1, i3b, i4b, i5b)
    fire3(i3b, i4b, i5b, ga0b, b4b, b5b, sgb)

    N4 = N_CHUNKS // 4
    true_ = jnp.bool_(True)

    def body(c2, carry):
        e = 4 * c2
        not_first = c2 > 0
        not_last = c2 < N4 - 1
        phase(e, i3a, i4a, i5a, ga0a, ga1a, b4a, b5a, sga, soa,
              not_first, true_)
        phase(e + 1, i3b, i4b, i5b, ga0b, ga1b, b4b, b5b, sgb, sob,
              not_first, true_)
        phase(e + 2, i3a, i4a, i5a, ga1a, ga0a, b4a, b5a, sga, soa,
              true_, not_last)
        phase(e + 3, i3b, i4b, i5b, ga1b, ga0b, b4b, b5b, sgb, sob,
              true_, not_last)
        return carry

    lax.fori_loop(0, N4, body, 0)
    waitsc(ga1a, soa)
    waitsc(ga1b, sob)


@jax.jit
def _run(bytes_i32, table_3, table_4, table_5):
    mesh = plsc.VectorSubcoreMesh(core_axis_name="c", subcore_axis_name="s")
    idx_t = pltpu.VMEM((CHUNK,), jnp.int32)
    row_t = pltpu.VMEM((CHUNK, HIDDEN), jnp.float32)
    k = pl.kernel(
        _hash_body,
        out_type=jax.ShapeDtypeStruct((TOK, HIDDEN), jnp.float32),
        mesh=mesh,
        scratch_types=[
            pltpu.VMEM((T_PER_W + 8,), jnp.int32),
            idx_t, idx_t, idx_t, row_t, row_t, row_t, row_t,
            idx_t, idx_t, idx_t, row_t, row_t, row_t, row_t,
            pltpu.SemaphoreType.DMA, pltpu.SemaphoreType.DMA,
            pltpu.SemaphoreType.DMA, pltpu.SemaphoreType.DMA,
        ],
    )
    out = k(bytes_i32.reshape(TOK), table_3, table_4, table_5)
    return out.reshape(BATCH, SEQ, HIDDEN)


def kernel(bytes_seq, table_3, table_4, table_5):
    return _run(bytes_seq.astype(jnp.int32), table_3, table_4, table_5)


# speedup vs baseline: 1.1160x; 1.1160x over previous
import functools

import jax
import jax.numpy as jnp
from jax import lax
from jax.experimental import pallas as pl
from jax.experimental.pallas import tpu as pltpu
from jax.experimental.pallas import tpu_sc as plsc

NUM_HASHES = 50000
HIDDEN = 768
PRIME = 31
BATCH = 4
SEQ = 8192

NC = 2
NS = 16
NW = NC * NS
TOK = BATCH * SEQ
T_PER_W = TOK // NW
CHUNK = 16
N_CHUNKS = T_PER_W // CHUNK
N2 = N_CHUNKS // 2
W_PER_ROW = SEQ // T_PER_W
VPR = HIDDEN // 16


def _hash_body(bytes_hbm, t3, t4, t5, out_hbm, vb,
               i3a, i4a, i5a, ga0a, ga1a, b4a, b5a,
               i3b, i4b, i5b, ga0b, ga1b, b4b, b5b,
               sga, soa, sgb, sob):
    wid = lax.axis_index("s") * NC + lax.axis_index("c")
    p0 = (wid % W_PER_ROW) * T_PER_W
    f0 = wid * T_PER_W

    zero16 = jnp.zeros((16,), jnp.int32)
    vb[pl.ds(0, 16)] = zero16

    @pl.when(f0 == 0)
    def _():
        pltpu.sync_copy(bytes_hbm.at[pl.ds(0, T_PER_W)],
                        vb.at[pl.ds(8, T_PER_W)])

    @pl.when(f0 != 0)
    def _():
        pltpu.sync_copy(bytes_hbm.at[pl.ds(f0 - 8, T_PER_W + 8)], vb)

    lanes = lax.iota(jnp.int32, 16)

    def hash_chunk(c, i3, i4, i5):
        for g in range(CHUNK // 16):
            o = c * CHUNK + g * 16
            b0 = vb[pl.ds(8 + o, 16)]
            b1 = vb[pl.ds(7 + o, 16)]
            b2 = vb[pl.ds(6 + o, 16)]
            b3 = vb[pl.ds(5 + o, 16)]
            b4 = vb[pl.ds(4 + o, 16)]

            t_in_row = lanes + (p0 + o)

            h = b2 * PRIME + b1
            h3 = (h * PRIME + b0) % NUM_HASHES
            h3 = jnp.where(t_in_row >= 2, h3, 0)

            h = (b3 * PRIME + b2)
            h = (h * PRIME + b1) % NUM_HASHES
            h4 = (h * PRIME + b0) % NUM_HASHES
            h4 = jnp.where(t_in_row >= 3, h4, 0)

            h = (b4 * PRIME + b3)
            h = (h * PRIME + b2) % NUM_HASHES
            h = (h * PRIME + b1) % NUM_HASHES
            h5 = (h * PRIME + b0) % NUM_HASHES
            h5 = jnp.where(t_in_row >= 4, h5, 0)

            i3[pl.ds(g * 16, 16)] = h3
            i4[pl.ds(g * 16, 16)] = h4
            i5[pl.ds(g * 16, 16)] = h5

    def fire3(i3, i4, i5, acc, b4, b5, sg):
        pltpu.async_copy(t3.at[i3], acc, sg)
        pltpu.async_copy(t4.at[i4], b4, sg)
        pltpu.async_copy(t5.at[i5], b5, sg)

    def wait3(i3, acc, b4, b5, sg):
        pltpu.make_async_copy(t3.at[i3], acc, sg).wait()
        pltpu.make_async_copy(t3.at[i3], b4, sg).wait()
        pltpu.make_async_copy(t3.at[i3], b5, sg).wait()

    def add2(acc, b4, b5):
        def add_row(r, carry):
            for u in range(VPR):
                v = (b4[r, pl.ds(u * 16, 16)] +
                     b5[r, pl.ds(u * 16, 16)])
                plsc.addupdate(acc.at[r, pl.ds(u * 16, 16)], v)
            return carry
        lax.fori_loop(0, CHUNK, add_row, 0)

    def scatter(acc, so, c):
        pltpu.async_copy(acc, out_hbm.at[pl.ds(f0 + c * CHUNK, CHUNK)], so)

    def waitsc(acc, so):
        pltpu.make_async_copy(acc, out_hbm.at[pl.ds(0, CHUNK)], so).wait()

    def phase(c, i3, i4, i5, gacc, ganext, b4, b5, sg, so,
              do_waitsc, do_fire):
        wait3(i3, gacc, b4, b5, sg)
        add2(gacc, b4, b5)
        scatter(gacc, so, c)

        @pl.when(do_waitsc)
        def _():
            waitsc(gacc, so)

        @pl.when(do_fire)
        def _():
            hash_chunk(c + 2, i3, i4, i5)
            fire3(i3, i4, i5, ganext, b4, b5, sg)

    hash_chunk(0, i3a, i4a, i5a)
    fire3(i3a, i4a, i5a, ga0a, b4a, b5a, sga)
    hash_chunk(1, i3b, i4b, i5b)
    fire3(i3b, i4b, i5b, ga0b, b4b, b5b, sgb)

    N4 = N_CHUNKS // 4
    true_ = jnp.bool_(True)

    def body(c2, carry):
        e = 4 * c2
        not_first = c2 > 0
        not_last = c2 < N4 - 1
        phase(e, i3a, i4a, i5a, ga0a, ga1a, b4a, b5a, sga, soa,
              not_first, true_)
        phase(e + 1, i3b, i4b, i5b, ga0b, ga1b, b4b, b5b, sgb, sob,
              not_first, true_)
        phase(e + 2, i3a, i4a, i5a, ga1a, ga0a, b4a, b5a, sga, soa,
              true_, not_last)
        phase(e + 3, i3b, i4b, i5b, ga1b, ga0b, b4b, b5b, sgb, sob,
              true_, not_last)
        return carry

    lax.fori_loop(0, N4, body, 0)
    waitsc(ga1a, soa)
    waitsc(ga1b, sob)


@jax.jit
def _run(bytes_i32, table_3, table_4, table_5):
    mesh = plsc.VectorSubcoreMesh(core_axis_name="c", subcore_axis_name="s")
    idx_t = pltpu.VMEM((CHUNK,), jnp.int32)
    row_t = pltpu.VMEM((CHUNK, HIDDEN), jnp.float32)
    k = pl.kernel(
        _hash_body,
        out_type=jax.ShapeDtypeStruct((TOK, HIDDEN), jnp.float32),
        mesh=mesh,
        scratch_types=[
            pltpu.VMEM((T_PER_W + 8,), jnp.int32),
            idx_t, idx_t, idx_t, row_t, row_t, row_t, row_t,
            idx_t, idx_t, idx_t, row_t, row_t, row_t, row_t,
            pltpu.SemaphoreType.DMA, pltpu.SemaphoreType.DMA,
            pltpu.SemaphoreType.DMA, pltpu.SemaphoreType.DMA,
        ],
    )
    out = k(bytes_i32.reshape(TOK), table_3, table_4, table_5)
    return out.reshape(BATCH, SEQ, HIDDEN)


def kernel(bytes_seq, table_3, table_4, table_5):
    return _run(bytes_seq.astype(jnp.int32), table_3, table_4, table_5)


# speedup vs baseline: 1.2706x; 1.1386x over previous
import functools

import jax
import jax.numpy as jnp
from jax import lax
from jax.experimental import pallas as pl
from jax.experimental.pallas import tpu as pltpu
from jax.experimental.pallas import tpu_sc as plsc

NUM_HASHES = 50000
HIDDEN = 768
PRIME = 31
BATCH = 4
SEQ = 8192

NC = 2
NS = 16
NW = NC * NS
TOK = BATCH * SEQ
T_PER_W = TOK // NW
CHUNK = 16
N_CHUNKS = T_PER_W // CHUNK
N2 = N_CHUNKS // 2
W_PER_ROW = SEQ // T_PER_W
VPR = HIDDEN // 16


def _hash_body(bytes_hbm, t3, t4, t5, out_hbm, vb,
               i3a, i4a, i5a, ga0a, ga1a, b4a, b5a,
               i3b, i4b, i5b, ga0b, ga1b, b4b, b5b,
               sga, soa, sgb, sob):
    wid = lax.axis_index("s") * NC + lax.axis_index("c")
    p0 = (wid % W_PER_ROW) * T_PER_W
    f0 = wid * T_PER_W

    zero16 = jnp.zeros((16,), jnp.int32)
    vb[pl.ds(0, 16)] = zero16

    @pl.when(f0 == 0)
    def _():
        pltpu.sync_copy(bytes_hbm.at[pl.ds(0, T_PER_W)],
                        vb.at[pl.ds(8, T_PER_W)])

    @pl.when(f0 != 0)
    def _():
        pltpu.sync_copy(bytes_hbm.at[pl.ds(f0 - 8, T_PER_W + 8)], vb)

    lanes = lax.iota(jnp.int32, 16)

    def hash_chunk(c, i3, i4, i5):
        for g in range(CHUNK // 16):
            o = c * CHUNK + g * 16
            b0 = vb[pl.ds(8 + o, 16)]
            b1 = vb[pl.ds(7 + o, 16)]
            b2 = vb[pl.ds(6 + o, 16)]
            b3 = vb[pl.ds(5 + o, 16)]
            b4 = vb[pl.ds(4 + o, 16)]

            t_in_row = lanes + (p0 + o)

            h = b2 * PRIME + b1
            h3 = (h * PRIME + b0) % NUM_HASHES
            h3 = jnp.where(t_in_row >= 2, h3, 0)

            h = (b3 * PRIME + b2)
            h = (h * PRIME + b1) % NUM_HASHES
            h4 = (h * PRIME + b0) % NUM_HASHES
            h4 = jnp.where(t_in_row >= 3, h4, 0)

            h = (b4 * PRIME + b3)
            h = (h * PRIME + b2) % NUM_HASHES
            h = (h * PRIME + b1) % NUM_HASHES
            h5 = (h * PRIME + b0) % NUM_HASHES
            h5 = jnp.where(t_in_row >= 4, h5, 0)

            i3[pl.ds(g * 16, 16)] = h3
            i4[pl.ds(g * 16, 16)] = h4
            i5[pl.ds(g * 16, 16)] = h5

    def fire3(i3, i4, i5, acc, b4, b5, sg):
        pltpu.async_copy(t3.at[i3], acc, sg)
        pltpu.async_copy(t4.at[i4], b4, sg)
        pltpu.async_copy(t5.at[i5], b5, sg)

    def wait3(i3, acc, b4, b5, sg):
        pltpu.make_async_copy(t3.at[i3], acc, sg).wait()
        pltpu.make_async_copy(t3.at[i3], b4, sg).wait()
        pltpu.make_async_copy(t3.at[i3], b5, sg).wait()

    def add2(acc, b4, b5):
        def add_rows(r2, carry):
            for d in range(2):
                r = r2 * 2 + d
                for u in range(VPR):
                    v = (b4[r, pl.ds(u * 16, 16)] +
                         b5[r, pl.ds(u * 16, 16)])
                    plsc.addupdate(acc.at[r, pl.ds(u * 16, 16)], v)
            return carry
        lax.fori_loop(0, CHUNK // 2, add_rows, 0)

    def scatter(acc, so, c):
        pltpu.async_copy(acc, out_hbm.at[pl.ds(f0 + c * CHUNK, CHUNK)], so)

    def waitsc(acc, so):
        pltpu.make_async_copy(acc, out_hbm.at[pl.ds(0, CHUNK)], so).wait()

    def phase(c, i3, i4, i5, gacc, ganext, b4, b5, sg, so,
              do_waitsc, do_fire):
        wait3(i3, gacc, b4, b5, sg)
        add2(gacc, b4, b5)
        scatter(gacc, so, c)

        @pl.when(do_waitsc)
        def _():
            waitsc(gacc, so)

        @pl.when(do_fire)
        def _():
            hash_chunk(c + 2, i3, i4, i5)
            fire3(i3, i4, i5, ganext, b4, b5, sg)

    hash_chunk(0, i3a, i4a, i5a)
    fire3(i3a, i4a, i5a, ga0a, b4a, b5a, sga)
    hash_chunk(1, i3b, i4b, i5b)
    fire3(i3b, i4b, i5b, ga0b, b4b, b5b, sgb)

    N4 = N_CHUNKS // 4
    true_ = jnp.bool_(True)

    def body(c2, carry):
        e = 4 * c2
        not_first = c2 > 0
        not_last = c2 < N4 - 1
        phase(e, i3a, i4a, i5a, ga0a, ga1a, b4a, b5a, sga, soa,
              not_first, true_)
        phase(e + 1, i3b, i4b, i5b, ga0b, ga1b, b4b, b5b, sgb, sob,
              not_first, true_)
        phase(e + 2, i3a, i4a, i5a, ga1a, ga0a, b4a, b5a, sga, soa,
              true_, not_last)
        phase(e + 3, i3b, i4b, i5b, ga1b, ga0b, b4b, b5b, sgb, sob,
              true_, not_last)
        return carry

    lax.fori_loop(0, N4, body, 0)
    waitsc(ga1a, soa)
    waitsc(ga1b, sob)


@jax.jit
def _run(bytes_i32, table_3, table_4, table_5):
    mesh = plsc.VectorSubcoreMesh(core_axis_name="c", subcore_axis_name="s")
    idx_t = pltpu.VMEM((CHUNK,), jnp.int32)
    row_t = pltpu.VMEM((CHUNK, HIDDEN), jnp.float32)
    k = pl.kernel(
        _hash_body,
        out_type=jax.ShapeDtypeStruct((TOK, HIDDEN), jnp.float32),
        mesh=mesh,
        scratch_types=[
            pltpu.VMEM((T_PER_W + 8,), jnp.int32),
            idx_t, idx_t, idx_t, row_t, row_t, row_t, row_t,
            idx_t, idx_t, idx_t, row_t, row_t, row_t, row_t,
            pltpu.SemaphoreType.DMA, pltpu.SemaphoreType.DMA,
            pltpu.SemaphoreType.DMA, pltpu.SemaphoreType.DMA,
        ],
    )
    out = k(bytes_i32.reshape(TOK), table_3, table_4, table_5)
    return out.reshape(BATCH, SEQ, HIDDEN)


def kernel(bytes_seq, table_3, table_4, table_5):
    return _run(bytes_seq.astype(jnp.int32), table_3, table_4, table_5)
